# baseline (speedup 1.0000x reference)
import functools

import jax
import jax.numpy as jnp
from jax import lax
from jax.experimental import pallas as pl
from jax.experimental.pallas import tpu as pltpu
from jax.experimental.pallas import tpu_sc as plsc

E = 3200000
NW = 32
C = 640
SUB = 128
NSUB = C // SUB
NCHUNK = E // C
FP = 16
SEG = 8
F_XS, F_XT, F_E, F_U = 10, 5, 10, 10
N_GRAPHS = 64
BLK = 8000
NSB = E // (SEG * BLK)
CP = C // SEG


def _sc_gather(src2d, tgt2d, b2d, xs_t, xt_t, u_t):
    mesh = plsc.VectorSubcoreMesh(core_axis_name="c", subcore_axis_name="s")

    @functools.partial(
        pl.kernel,
        mesh=mesh,
        out_type=jax.ShapeDtypeStruct((E // SEG, SEG, FP), jnp.float32),
        scratch_types=[
            pltpu.VMEM((C,), jnp.int32),
            pltpu.VMEM((C,), jnp.int32),
            pltpu.VMEM((C,), jnp.int32),
            pltpu.VMEM((C, FP), jnp.float32),
            pltpu.VMEM((C, FP), jnp.float32),
            pltpu.VMEM((C, FP), jnp.float32),
            pltpu.VMEM((CP, SEG, FP), jnp.float32),
            pltpu.SemaphoreType.DMA,
        ],
        compiler_params=pltpu.CompilerParams(use_tc_tiling_on_sc=False),
    )
    def body(src_hbm, tgt_hbm, b_hbm, xs_hbm, xt_hbm, u_hbm, g_hbm,
             src_v, tgt_v, b_v, gs_v, gt_v, gu_v, gp_v, sem):
        w = lax.axis_index("s") * 2 + lax.axis_index("c")
        n_w = (NCHUNK - w + NW - 1) // NW

        def chunk(i, carry):
            k = w + i * NW
            pltpu.sync_copy(src_hbm.at[k], src_v)
            pltpu.sync_copy(tgt_hbm.at[k], tgt_v)
            pltpu.sync_copy(b_hbm.at[k], b_v)
            copies = []
            for j in range(NSUB):
                s = j * SUB
                copies.append(pltpu.async_copy(
                    xs_hbm.at[src_v.at[pl.ds(s, SUB)]],
                    gs_v.at[pl.ds(s, SUB)], sem))
                copies.append(pltpu.async_copy(
                    xt_hbm.at[tgt_v.at[pl.ds(s, SUB)]],
                    gt_v.at[pl.ds(s, SUB)], sem))
                copies.append(pltpu.async_copy(
                    u_hbm.at[b_v.at[pl.ds(s, SUB)]],
                    gu_v.at[pl.ds(s, SUB)], sem))
            for cp in copies:
                cp.wait()

            def merge(r, carry2):
                row = r * SEG
                for e in range(SEG):
                    gp_v[r, e] = (gs_v[row + e] + gt_v[row + e]
                                  + gu_v[row + e])
                return carry2

            lax.fori_loop(0, CP, merge, 0)
            pltpu.sync_copy(gp_v, g_hbm.at[pl.ds(k * CP, CP)])
            return carry

        lax.fori_loop(0, n_w, chunk, 0)

    return body(src2d, tgt2d, b2d, xs_t, xt_t, u_t)


def _tc_body(g_ref, ea_ref, w1_ref, w2_ref, b2_ref, o_ref):
    s = pl.program_id(1)
    rows = lax.broadcasted_iota(jnp.int32, (SEG * FP, FP), 0)
    cols = lax.broadcasted_iota(jnp.int32, (SEG * FP, FP), 1)
    sel = (rows == s * FP + cols).astype(jnp.float32)
    z = jnp.dot(g_ref[...], sel, preferred_element_type=jnp.float32)
    z = z + jnp.dot(ea_ref[...], w1_ref[...], preferred_element_type=jnp.float32)
    h1 = jnp.where(z >= 0, z, 0.1 * z)
    o_ref[...] = (jnp.dot(h1, w2_ref[...], preferred_element_type=jnp.float32)
                  + b2_ref[...])


def _tc_mlp(g2, ea, w1et, w2tp, b2r):
    return pl.pallas_call(
        _tc_body,
        grid=(NSB, SEG),
        in_specs=[
            pl.BlockSpec((BLK, SEG * FP), lambda r, s: (r, 0)),
            pl.BlockSpec((BLK, F_E), lambda r, s: (r * SEG + s, 0)),
            pl.BlockSpec((F_E, FP), lambda r, s: (0, 0)),
            pl.BlockSpec((FP, F_E), lambda r, s: (0, 0)),
            pl.BlockSpec((1, F_E), lambda r, s: (0, 0)),
        ],
        out_specs=pl.BlockSpec((BLK, F_E), lambda r, s: (r * SEG + s, 0)),
        out_shape=jax.ShapeDtypeStruct((E, F_E), jnp.float32),
    )(g2, ea, w1et, w2tp, b2r)


def _perm(a):
    return a.reshape(NSB, SEG, BLK).transpose(0, 2, 1).reshape(NCHUNK, C)


def kernel(x_s, x_t, edge_index, edge_attr, u, batch_e, W1, b1, W2, b2):
    src2d = _perm(edge_index[0])
    tgt2d = _perm(edge_index[1])
    b2d = _perm(batch_e)

    xs1 = x_s @ W1[:, :F_XS].T
    xt1 = x_t @ W1[:, F_XS:F_XS + F_XT].T
    u1 = u @ W1[:, F_XS + F_XT + F_E:].T + b1

    def padw(a):
        return jnp.pad(a, ((0, 0), (0, FP - a.shape[1])))

    g3 = _sc_gather(src2d, tgt2d, b2d, padw(xs1), padw(xt1), padw(u1))
    g2 = g3.reshape(E // SEG, SEG * FP)

    w1et = padw(W1[:, F_XS + F_XT:F_XS + F_XT + F_E].T)
    w2tp = jnp.pad(W2.T, ((0, FP - F_E), (0, 0)))
    return _tc_mlp(g2, edge_attr, w1et, w2tp, b2.reshape(1, F_E))


# speedup vs baseline: 2.2960x; 2.2960x over previous
import functools

import jax
import jax.numpy as jnp
from jax import lax
from jax.experimental import pallas as pl
from jax.experimental.pallas import tpu as pltpu
from jax.experimental.pallas import tpu_sc as plsc

E = 3200000
NW = 32
C = 640
SUB = 128
NSUB = C // SUB
NCHUNK = E // C
FP = 16
SEG = 8
F_XS, F_XT, F_E, F_U = 10, 5, 10, 10
N_GRAPHS = 64
BLK = 8000
NSB = E // (SEG * BLK)
CP = C // SEG


def _sc_gather(src2d, tgt2d, xs_t, xt_t):
    mesh = plsc.VectorSubcoreMesh(core_axis_name="c", subcore_axis_name="s")

    @functools.partial(
        pl.kernel,
        mesh=mesh,
        out_type=jax.ShapeDtypeStruct((E // SEG, SEG, FP), jnp.float32),
        scratch_types=[
            pltpu.VMEM((C,), jnp.int32),
            pltpu.VMEM((C,), jnp.int32),
            pltpu.VMEM((C, FP), jnp.float32),
            pltpu.VMEM((C, FP), jnp.float32),
            pltpu.VMEM((CP, SEG, FP), jnp.float32),
            pltpu.SemaphoreType.DMA,
        ],
        compiler_params=pltpu.CompilerParams(use_tc_tiling_on_sc=False),
    )
    def body(src_hbm, tgt_hbm, xs_hbm, xt_hbm, g_hbm,
             src_v, tgt_v, gs_v, gt_v, gp_v, sem):
        w = lax.axis_index("s") * 2 + lax.axis_index("c")
        n_w = (NCHUNK - w + NW - 1) // NW

        def chunk(i, carry):
            k = w + i * NW
            pltpu.sync_copy(src_hbm.at[k], src_v)
            pltpu.sync_copy(tgt_hbm.at[k], tgt_v)

            copies = []
            for j in range(NSUB):
                s = j * SUB
                copies.append(pltpu.async_copy(
                    xs_hbm.at[src_v.at[pl.ds(s, SUB)]],
                    gs_v.at[pl.ds(s, SUB)], sem))
                copies.append(pltpu.async_copy(
                    xt_hbm.at[tgt_v.at[pl.ds(s, SUB)]],
                    gt_v.at[pl.ds(s, SUB)], sem))

            for cp in copies:
                cp.wait()

            def merge(r, carry2):
                row = r * SEG
                for e in range(SEG):
                    gp_v[r, e] = gs_v[row + e] + gt_v[row + e]
                return carry2

            lax.fori_loop(0, CP, merge, 0)
            pltpu.sync_copy(gp_v, g_hbm.at[pl.ds(k * CP, CP)])
            return carry

        lax.fori_loop(0, n_w, chunk, 0)

    return body(src2d, tgt2d, xs_t, xt_t)


def _tc_body(g_ref, ea_ref, b_ref, u1_ref, w1_ref, w2_ref, b2_ref, o_ref):
    s = pl.program_id(1)
    bcol = b_ref[0]
    onehot = (bcol == lax.broadcasted_iota(jnp.int32, (BLK, N_GRAPHS), 1)
              ).astype(jnp.float32)
    rows = lax.broadcasted_iota(jnp.int32, (SEG * FP, FP), 0)
    cols = lax.broadcasted_iota(jnp.int32, (SEG * FP, FP), 1)
    sel = (rows == s * FP + cols).astype(jnp.float32)
    z = jnp.dot(g_ref[...], sel, preferred_element_type=jnp.float32)
    z = z + jnp.dot(onehot, u1_ref[...], preferred_element_type=jnp.float32)
    z = z + jnp.dot(ea_ref[...], w1_ref[...], preferred_element_type=jnp.float32)
    h1 = jnp.where(z >= 0, z, 0.1 * z)
    o_ref[...] = (jnp.dot(h1, w2_ref[...], preferred_element_type=jnp.float32)
                  + b2_ref[...])


def _tc_mlp(g2, ea, batch3, u1p, w1et, w2tp, b2r):
    return pl.pallas_call(
        _tc_body,
        grid=(NSB, SEG),
        in_specs=[
            pl.BlockSpec((BLK, SEG * FP), lambda r, s: (r, 0)),
            pl.BlockSpec((BLK, F_E), lambda r, s: (r * SEG + s, 0)),
            pl.BlockSpec((1, BLK, 1), lambda r, s: (r * SEG + s, 0, 0)),
            pl.BlockSpec((N_GRAPHS, FP), lambda r, s: (0, 0)),
            pl.BlockSpec((F_E, FP), lambda r, s: (0, 0)),
            pl.BlockSpec((FP, F_E), lambda r, s: (0, 0)),
            pl.BlockSpec((1, F_E), lambda r, s: (0, 0)),
        ],
        out_specs=pl.BlockSpec((BLK, F_E), lambda r, s: (r * SEG + s, 0)),
        out_shape=jax.ShapeDtypeStruct((E, F_E), jnp.float32),
    )(g2, ea, batch3, u1p, w1et, w2tp, b2r)


def _perm(a):
    return a.reshape(NSB, SEG, BLK).transpose(0, 2, 1).reshape(NCHUNK, C)


def kernel(x_s, x_t, edge_index, edge_attr, u, batch_e, W1, b1, W2, b2):
    src2d = _perm(edge_index[0])
    tgt2d = _perm(edge_index[1])

    xs1 = x_s @ W1[:, :F_XS].T
    xt1 = x_t @ W1[:, F_XS:F_XS + F_XT].T
    u1 = u @ W1[:, F_XS + F_XT + F_E:].T + b1

    def padw(a):
        return jnp.pad(a, ((0, 0), (0, FP - a.shape[1])))

    g3 = _sc_gather(src2d, tgt2d, padw(xs1), padw(xt1))
    g2 = g3.reshape(E // SEG, SEG * FP)
    batch3 = batch_e.reshape(E // BLK, BLK, 1)
    u1p = padw(u1)

    w1et = padw(W1[:, F_XS + F_XT:F_XS + F_XT + F_E].T)
    w2tp = jnp.pad(W2.T, ((0, FP - F_E), (0, 0)))
    return _tc_mlp(g2, edge_attr, batch3, u1p, w1et, w2tp,
                   b2.reshape(1, F_E))


# speedup vs baseline: 4.4011x; 1.9168x over previous
import functools

import jax
import jax.numpy as jnp
from jax import lax
from jax.experimental import pallas as pl
from jax.experimental.pallas import tpu as pltpu
from jax.experimental.pallas import tpu_sc as plsc

E = 3200000
NW = 32
C = 640
SUB = 128
NSUB = C // SUB
NCHUNK = E // C
FP = 16
SEG = 8
F_XS, F_XT, F_E, F_U = 10, 5, 10, 10
N_GRAPHS = 64
BLK = 8000
NSB = E // (SEG * BLK)


def _sc_gather(src2d, tgt2d, xs_t, xt_t):
    mesh = plsc.VectorSubcoreMesh(core_axis_name="c", subcore_axis_name="s")

    @functools.partial(
        pl.kernel,
        mesh=mesh,
        out_type=(
            jax.ShapeDtypeStruct((NCHUNK, C, FP), jnp.float32),
            jax.ShapeDtypeStruct((NCHUNK, C, FP), jnp.float32),
        ),
        scratch_types=[
            pltpu.VMEM((C,), jnp.int32),
            pltpu.VMEM((C,), jnp.int32),
            pltpu.VMEM((C, FP), jnp.float32),
            pltpu.VMEM((C, FP), jnp.float32),
            pltpu.SemaphoreType.DMA,
        ],
        compiler_params=pltpu.CompilerParams(use_tc_tiling_on_sc=False),
    )
    def body(src_hbm, tgt_hbm, xs_hbm, xt_hbm, gs_hbm, gt_hbm,
             src_v, tgt_v, gs_v, gt_v, sem):
        w = lax.axis_index("s") * 2 + lax.axis_index("c")
        n_w = (NCHUNK - w + NW - 1) // NW

        def chunk(i, carry):
            k = w + i * NW
            pltpu.sync_copy(src_hbm.at[k], src_v)
            pltpu.sync_copy(tgt_hbm.at[k], tgt_v)
            copies = []
            for j in range(NSUB):
                s = j * SUB
                copies.append(pltpu.async_copy(
                    xs_hbm.at[src_v.at[pl.ds(s, SUB)]],
                    gs_v.at[pl.ds(s, SUB)], sem))
                copies.append(pltpu.async_copy(
                    xt_hbm.at[tgt_v.at[pl.ds(s, SUB)]],
                    gt_v.at[pl.ds(s, SUB)], sem))
            for cp in copies:
                cp.wait()
            pltpu.sync_copy(gs_v, gs_hbm.at[k])
            pltpu.sync_copy(gt_v, gt_hbm.at[k])
            return carry

        lax.fori_loop(0, n_w, chunk, 0)

    return body(src2d, tgt2d, xs_t, xt_t)


def _tc_body(gs_ref, gt_ref, ea_ref, st_ref, en_ref, u1_ref, w1_ref, w2_ref,
             b2_ref, o_ref):
    r = pl.program_id(0)
    s = pl.program_id(1)
    rows = lax.broadcasted_iota(jnp.int32, (SEG * FP, FP), 0)
    cols = lax.broadcasted_iota(jnp.int32, (SEG * FP, FP), 1)
    sel = (rows == s * FP + cols).astype(jnp.float32)
    z = jnp.dot(gs_ref[...] + gt_ref[...], sel,
                preferred_element_type=jnp.float32)
    base = (r * SEG + s) * BLK
    eidx = base + lax.broadcasted_iota(jnp.int32, (BLK, N_GRAPHS), 0)
    onehot = ((eidx >= st_ref[...]) & (eidx < en_ref[...])).astype(jnp.float32)
    z = z + jnp.dot(onehot, u1_ref[...], preferred_element_type=jnp.float32)
    z = z + jnp.dot(ea_ref[...], w1_ref[...], preferred_element_type=jnp.float32)
    h1 = jnp.where(z >= 0, z, 0.1 * z)
    o_ref[...] = (jnp.dot(h1, w2_ref[...], preferred_element_type=jnp.float32)
                  + b2_ref[...])


def _tc_mlp(gs2, gt2, ea, starts, ends, u1p, w1et, w2tp, b2r):
    return pl.pallas_call(
        _tc_body,
        grid=(NSB, SEG),
        in_specs=[
            pl.BlockSpec((BLK, SEG * FP), lambda r, s: (r, 0)),
            pl.BlockSpec((BLK, SEG * FP), lambda r, s: (r, 0)),
            pl.BlockSpec((BLK, F_E), lambda r, s: (r * SEG + s, 0)),
            pl.BlockSpec((1, N_GRAPHS), lambda r, s: (0, 0)),
            pl.BlockSpec((1, N_GRAPHS), lambda r, s: (0, 0)),
            pl.BlockSpec((N_GRAPHS, FP), lambda r, s: (0, 0)),
            pl.BlockSpec((F_E, FP), lambda r, s: (0, 0)),
            pl.BlockSpec((FP, F_E), lambda r, s: (0, 0)),
            pl.BlockSpec((1, F_E), lambda r, s: (0, 0)),
        ],
        out_specs=pl.BlockSpec((BLK, F_E), lambda r, s: (r * SEG + s, 0)),
        out_shape=jax.ShapeDtypeStruct((E, F_E), jnp.float32),
    )(gs2, gt2, ea, starts, ends, u1p, w1et, w2tp, b2r)


def _perm(a):
    return a.reshape(NSB, SEG, BLK).transpose(0, 2, 1).reshape(NCHUNK, C)


def kernel(x_s, x_t, edge_index, edge_attr, u, batch_e, W1, b1, W2, b2):
    src2d = _perm(edge_index[0])
    tgt2d = _perm(edge_index[1])

    xs1 = x_s @ W1[:, :F_XS].T
    xt1 = x_t @ W1[:, F_XS:F_XS + F_XT].T
    u1 = u @ W1[:, F_XS + F_XT + F_E:].T + b1

    def padw(a):
        return jnp.pad(a, ((0, 0), (0, FP - a.shape[1])))

    gs3, gt3 = _sc_gather(src2d, tgt2d, padw(xs1), padw(xt1))
    gs2 = gs3.reshape(E // SEG, SEG * FP)
    gt2 = gt3.reshape(E // SEG, SEG * FP)

    ss = jnp.searchsorted(
        batch_e, jnp.arange(N_GRAPHS + 1, dtype=batch_e.dtype)).astype(jnp.int32)
    starts = ss[:N_GRAPHS].reshape(1, N_GRAPHS)
    ends = ss[1:].reshape(1, N_GRAPHS)

    w1et = padw(W1[:, F_XS + F_XT:F_XS + F_XT + F_E].T)
    w2tp = jnp.pad(W2.T, ((0, FP - F_E), (0, 0)))
    return _tc_mlp(gs2, gt2, edge_attr, starts, ends, padw(u1), w1et, w2tp,
                   b2.reshape(1, F_E))
